# Initial kernel scaffold; baseline (speedup 1.0000x reference)
import jax
import jax.numpy as jnp
from jax.experimental import pallas as pl


def kernel(x, embedding):
    raise NotImplementedError("write your pallas kernel here")


# speedup vs baseline: 1.2235x; 1.2235x over previous
import functools

import jax
import jax.numpy as jnp
from jax import lax
from jax.experimental import pallas as pl
from jax.experimental.pallas import tpu as pltpu
from jax.experimental.pallas import tpu_sc as plsc

N_EMB = 8192
DIM = 256
N_TOK = 4096
TOK_TILE = 512
N_TOK_TILES = N_TOK // TOK_TILE
CODE_CHUNK = 1024
N_CODE_CHUNKS = N_EMB // CODE_CHUNK

OH_TILE_M = 2048
N_OH_J = N_EMB // OH_TILE_M


_SC_WORKERS = 32
_TOK_PER_W = N_TOK // _SC_WORKERS


def _codes_body(x_ref, xs_ref, es_ref, emb_ref, codes_ref):
    xt = x_ref[...].astype(jnp.bfloat16)
    xs = xs_ref[0, 0, :]
    rmin = jnp.full((TOK_TILE,), jnp.inf, jnp.float32)
    ridx = jnp.zeros((TOK_TILE,), jnp.int32)
    for k in range(N_CODE_CHUNKS):
        et = emb_ref[pl.ds(k * CODE_CHUNK, CODE_CHUNK), :].astype(jnp.bfloat16)
        mm = lax.dot_general(
            xt, et, (((1,), (1,)), ((), ())),
            preferred_element_type=jnp.float32,
        )
        esk = es_ref[0, pl.ds(k * CODE_CHUNK, CODE_CHUNK)]
        t = xs[:, None] + esk[None, :]
        d = t - 2.0 * mm
        lmin = jnp.min(d, axis=1)
        iot = lax.broadcasted_iota(jnp.int32, (TOK_TILE, CODE_CHUNK), 1)
        lidx = jnp.min(
            jnp.where(d == lmin[:, None], iot, jnp.int32(2**30)), axis=1
        ) + k * CODE_CHUNK
        upd = lmin < rmin
        rmin = jnp.where(upd, lmin, rmin)
        ridx = jnp.where(upd, lidx, ridx)
    codes_ref[0, 0, :] = ridx


def _codes_call(x_flat, xs3, es2, emb):
    return pl.pallas_call(
        _codes_body,
        grid=(N_TOK_TILES,),
        in_specs=[
            pl.BlockSpec((TOK_TILE, DIM), lambda i: (i, 0)),
            pl.BlockSpec((1, 1, TOK_TILE), lambda i: (i, 0, 0)),
            pl.BlockSpec((1, N_EMB), lambda i: (0, 0)),
            pl.BlockSpec((N_EMB, DIM), lambda i: (0, 0)),
        ],
        out_specs=pl.BlockSpec((1, 1, TOK_TILE), lambda i: (i, 0, 0)),
        out_shape=jax.ShapeDtypeStruct((N_TOK_TILES, 1, TOK_TILE), jnp.int32),
    )(x_flat, xs3, es2, emb)


def _onehot_body(codes_ref, oh_ref, counts_ref):
    j = pl.program_id(0)
    i = pl.program_id(1)
    c = codes_ref[0, 0, :]
    iot = lax.broadcasted_iota(jnp.int32, (TOK_TILE, OH_TILE_M), 1) \
        + j * OH_TILE_M
    oh = (c[:, None] == iot).astype(jnp.float32)
    oh_ref[...] = oh

    @pl.when(i == 0)
    def _():
        counts_ref[...] = jnp.zeros((1, OH_TILE_M), jnp.float32)

    counts_ref[...] += jnp.sum(oh, axis=0)[None, :]


def _onehot_call(codes3):
    return pl.pallas_call(
        _onehot_body,
        grid=(N_OH_J, N_TOK_TILES),
        in_specs=[
            pl.BlockSpec((1, 1, TOK_TILE), lambda j, i: (i, 0, 0)),
        ],
        out_specs=[
            pl.BlockSpec((TOK_TILE, OH_TILE_M), lambda j, i: (i, j)),
            pl.BlockSpec((1, OH_TILE_M), lambda j, i: (0, j)),
        ],
        out_shape=[
            jax.ShapeDtypeStruct((N_TOK, N_EMB), jnp.float32),
            jax.ShapeDtypeStruct((1, N_EMB), jnp.float32),
        ],
    )(codes3)


def _gather_body(emb_hbm, codes_hbm, x_hbm, out_hbm, lp_hbm,
                 idx_v, q_v, x_v, o_v, lv, sem):
    wid = lax.axis_index("s") * 2 + lax.axis_index("c")
    base = wid * _TOK_PER_W
    pltpu.sync_copy(codes_hbm.at[pl.ds(base, _TOK_PER_W)], idx_v)
    pltpu.async_copy(emb_hbm.at[idx_v], q_v, sem).wait()
    pltpu.sync_copy(x_hbm.at[pl.ds(base, _TOK_PER_W)], x_v)

    def body(t, acc):
        for cc in range(DIM // 16):
            sl = pl.ds(cc * 16, 16)
            xq = x_v[t, sl]
            qq = q_v[t, sl]
            dd = qq - xq
            o_v[t, sl] = xq + dd
            acc = acc + dd * dd
        return acc

    acc = lax.fori_loop(0, _TOK_PER_W, body, jnp.zeros((16,), jnp.float32))
    lv[...] = acc
    pltpu.sync_copy(o_v, out_hbm.at[pl.ds(base, _TOK_PER_W)])
    pltpu.sync_copy(lv, lp_hbm.at[wid])


def _gather_call(emb, codes_flat, x_flat):
    mesh = plsc.VectorSubcoreMesh(core_axis_name="c", subcore_axis_name="s")
    fn = pl.kernel(
        _gather_body,
        out_type=[
            jax.ShapeDtypeStruct((N_TOK, DIM), jnp.float32),
            jax.ShapeDtypeStruct((_SC_WORKERS, 16), jnp.float32),
        ],
        mesh=mesh,
        scratch_types=[
            pltpu.VMEM((_TOK_PER_W,), jnp.int32),
            pltpu.VMEM((_TOK_PER_W, DIM), jnp.float32),
            pltpu.VMEM((_TOK_PER_W, DIM), jnp.float32),
            pltpu.VMEM((_TOK_PER_W, DIM), jnp.float32),
            pltpu.VMEM((16,), jnp.float32),
            pltpu.SemaphoreType.DMA,
        ],
    )
    return fn(emb, codes_flat, x_flat)


def _scalar_body(counts_ref, lp_ref, loss_ref, perp_ref):
    s = jnp.sum(lp_ref[...])
    loss = 0.25 * (s * jnp.float32(1.0 / (N_TOK * DIM)))
    p = counts_ref[0, :] * jnp.float32(1.0 / N_TOK)
    ent = jnp.sum(p * jnp.log(p + 1e-10))
    loss_ref[...] = loss[None, None]
    perp_ref[...] = jnp.exp(-ent)[None, None]


def _scalar_call(counts, lp):
    return pl.pallas_call(
        _scalar_body,
        out_shape=[
            jax.ShapeDtypeStruct((1, 1), jnp.float32),
            jax.ShapeDtypeStruct((1, 1), jnp.float32),
        ],
    )(counts, lp)


def kernel(x, embedding):
    B, T, D = x.shape
    x_flat = x.reshape(-1, D)
    x_sq = jnp.sum(x_flat ** 2, axis=1, keepdims=True)
    e_sq = jnp.sum(embedding ** 2, axis=1)
    xs3 = x_sq.reshape(N_TOK_TILES, 1, TOK_TILE)
    es2 = e_sq.reshape(1, N_EMB)

    codes3 = _codes_call(x_flat, xs3, es2, embedding)
    codes_flat = codes3.reshape(N_TOK)

    one_hot_flat, counts = _onehot_call(codes3)
    quant_st_flat, lp = _gather_call(embedding, codes_flat, x_flat)
    loss2, perp2 = _scalar_call(counts, lp)

    quantized_st = quant_st_flat.reshape(B, T, D)
    codes = codes_flat.reshape(B, T)
    one_hot = one_hot_flat.reshape(B, T, N_EMB)
    return quantized_st, codes, one_hot, loss2[0, 0], perp2[0, 0]


# speedup vs baseline: 1.2492x; 1.0210x over previous
import functools

import jax
import jax.numpy as jnp
from jax import lax
from jax.experimental import pallas as pl
from jax.experimental.pallas import tpu as pltpu
from jax.experimental.pallas import tpu_sc as plsc

N_EMB = 8192
DIM = 256
N_TOK = 4096
TOK_TILE = 512
N_TOK_TILES = N_TOK // TOK_TILE
CODE_CHUNK = 1024
N_CODE_CHUNKS = N_EMB // CODE_CHUNK

OH_TILE_M = 2048
N_OH_J = N_EMB // OH_TILE_M


_SC_WORKERS = 32
_TOK_PER_W = N_TOK // _SC_WORKERS


def _codes_body(x_ref, xs_ref, es_ref, emb_ref, codes_ref):
    xt = x_ref[...]
    xs = xs_ref[0, 0, :]
    rmin = jnp.full((TOK_TILE,), jnp.inf, jnp.float32)
    ridx = jnp.zeros((TOK_TILE,), jnp.int32)
    for k in range(N_CODE_CHUNKS):
        et = emb_ref[pl.ds(k * CODE_CHUNK, CODE_CHUNK), :]
        mm2 = lax.dot_general(
            xt, et, (((1,), (1,)), ((), ())),
            preferred_element_type=jnp.float32,
        )
        esk = es_ref[0, pl.ds(k * CODE_CHUNK, CODE_CHUNK)]
        t = xs[:, None] + esk[None, :]
        d = t - mm2
        lmin = jnp.min(d, axis=1)
        iot = lax.broadcasted_iota(jnp.int32, (TOK_TILE, CODE_CHUNK), 1)
        lidx = jnp.min(
            jnp.where(d == lmin[:, None], iot, jnp.int32(2**30)), axis=1
        ) + k * CODE_CHUNK
        upd = lmin < rmin
        rmin = jnp.where(upd, lmin, rmin)
        ridx = jnp.where(upd, lidx, ridx)
    codes_ref[0, 0, :] = ridx


def _codes_call(x2b, xs3, es2, emb_b):
    return pl.pallas_call(
        _codes_body,
        grid=(N_TOK_TILES,),
        in_specs=[
            pl.BlockSpec((TOK_TILE, DIM), lambda i: (i, 0)),
            pl.BlockSpec((1, 1, TOK_TILE), lambda i: (i, 0, 0)),
            pl.BlockSpec((1, N_EMB), lambda i: (0, 0)),
            pl.BlockSpec((N_EMB, DIM), lambda i: (0, 0)),
        ],
        out_specs=pl.BlockSpec((1, 1, TOK_TILE), lambda i: (i, 0, 0)),
        out_shape=jax.ShapeDtypeStruct((N_TOK_TILES, 1, TOK_TILE), jnp.int32),
    )(x2b, xs3, es2, emb_b)


def _onehot_body(codes_ref, oh_ref, counts_ref):
    j = pl.program_id(0)
    i = pl.program_id(1)
    c = codes_ref[0, 0, :]
    iot = lax.broadcasted_iota(jnp.int32, (TOK_TILE, OH_TILE_M), 1) \
        + j * OH_TILE_M
    oh = (c[:, None] == iot).astype(jnp.float32)
    oh_ref[...] = oh

    @pl.when(i == 0)
    def _():
        counts_ref[...] = jnp.zeros((1, OH_TILE_M), jnp.float32)

    counts_ref[...] += jnp.sum(oh, axis=0)[None, :]


def _onehot_call(codes3):
    return pl.pallas_call(
        _onehot_body,
        grid=(N_OH_J, N_TOK_TILES),
        in_specs=[
            pl.BlockSpec((1, 1, TOK_TILE), lambda j, i: (i, 0, 0)),
        ],
        out_specs=[
            pl.BlockSpec((TOK_TILE, OH_TILE_M), lambda j, i: (i, j)),
            pl.BlockSpec((1, OH_TILE_M), lambda j, i: (0, j)),
        ],
        out_shape=[
            jax.ShapeDtypeStruct((N_TOK, N_EMB), jnp.float32),
            jax.ShapeDtypeStruct((1, N_EMB), jnp.float32),
        ],
    )(codes3)


def _gather_body(emb_hbm, codes_hbm, x_hbm, out_hbm, lp_hbm,
                 idx_v, q_v, x_v, o_v, lv, sem):
    wid = lax.axis_index("s") * 2 + lax.axis_index("c")
    base = wid * _TOK_PER_W
    pltpu.sync_copy(codes_hbm.at[pl.ds(base, _TOK_PER_W)], idx_v)
    pltpu.async_copy(emb_hbm.at[idx_v], q_v, sem).wait()
    pltpu.sync_copy(x_hbm.at[pl.ds(base, _TOK_PER_W)], x_v)

    def body(t, acc):
        for cc in range(DIM // 16):
            sl = pl.ds(cc * 16, 16)
            xq = x_v[t, sl]
            qq = q_v[t, sl]
            dd = qq - xq
            o_v[t, sl] = xq + dd
            acc = acc + dd * dd
        return acc

    acc = lax.fori_loop(0, _TOK_PER_W, body, jnp.zeros((16,), jnp.float32))
    lv[...] = acc
    pltpu.sync_copy(o_v, out_hbm.at[pl.ds(base, _TOK_PER_W)])
    pltpu.sync_copy(lv, lp_hbm.at[wid])


def _gather_call(emb, codes_flat, x_flat):
    mesh = plsc.VectorSubcoreMesh(core_axis_name="c", subcore_axis_name="s")
    fn = pl.kernel(
        _gather_body,
        out_type=[
            jax.ShapeDtypeStruct((N_TOK, DIM), jnp.float32),
            jax.ShapeDtypeStruct((_SC_WORKERS, 16), jnp.float32),
        ],
        mesh=mesh,
        scratch_types=[
            pltpu.VMEM((_TOK_PER_W,), jnp.int32),
            pltpu.VMEM((_TOK_PER_W, DIM), jnp.float32),
            pltpu.VMEM((_TOK_PER_W, DIM), jnp.float32),
            pltpu.VMEM((_TOK_PER_W, DIM), jnp.float32),
            pltpu.VMEM((16,), jnp.float32),
            pltpu.SemaphoreType.DMA,
        ],
    )
    return fn(emb, codes_flat, x_flat)


def _scalar_body(counts_ref, lp_ref, loss_ref, perp_ref):
    s = jnp.sum(lp_ref[...])
    loss = 0.25 * (s * jnp.float32(1.0 / (N_TOK * DIM)))
    p = counts_ref[0, :] * jnp.float32(1.0 / N_TOK)
    ent = jnp.sum(p * jnp.log(p + 1e-10))
    loss_ref[...] = loss[None, None]
    perp_ref[...] = jnp.exp(-ent)[None, None]


def _scalar_call(counts, lp):
    return pl.pallas_call(
        _scalar_body,
        out_shape=[
            jax.ShapeDtypeStruct((1, 1), jnp.float32),
            jax.ShapeDtypeStruct((1, 1), jnp.float32),
        ],
    )(counts, lp)


def kernel(x, embedding):
    B, T, D = x.shape
    x_flat = x.reshape(-1, D)
    x_sq = jnp.sum(x_flat ** 2, axis=1, keepdims=True)
    e_sq = jnp.sum(embedding ** 2, axis=1)
    xs3 = x_sq.reshape(N_TOK_TILES, 1, TOK_TILE)
    es2 = e_sq.reshape(1, N_EMB)
    x2b = (x_flat + x_flat).astype(jnp.bfloat16)
    emb_b = embedding.astype(jnp.bfloat16)

    codes3 = _codes_call(x2b, xs3, es2, emb_b)
    codes_flat = codes3.reshape(N_TOK)

    one_hot_flat, counts = _onehot_call(codes3)
    quant_st_flat, lp = _gather_call(embedding, codes_flat, x_flat)
    loss2, perp2 = _scalar_call(counts, lp)

    quantized_st = quant_st_flat.reshape(B, T, D)
    codes = codes_flat.reshape(B, T)
    one_hot = one_hot_flat.reshape(B, T, N_EMB)
    return quantized_st, codes, one_hot, loss2[0, 0], perp2[0, 0]


# speedup vs baseline: 1.3230x; 1.0591x over previous
import functools

import jax
import jax.numpy as jnp
from jax import lax
from jax.experimental import pallas as pl
from jax.experimental.pallas import tpu as pltpu
from jax.experimental.pallas import tpu_sc as plsc

N_EMB = 8192
DIM = 256
N_TOK = 4096
TOK_TILE = 1024
N_TOK_TILES = N_TOK // TOK_TILE
CODE_CHUNK = 2048
N_CODE_CHUNKS = N_EMB // CODE_CHUNK

OH_TILE_M = 2048
N_OH_J = N_EMB // OH_TILE_M


_SC_WORKERS = 32
_TOK_PER_W = N_TOK // _SC_WORKERS


def _codes_body(x_ref, xs_ref, es_ref, emb_ref, codes_ref):
    xt = x_ref[...]
    xs = xs_ref[0, 0, :]
    rmin = jnp.full((TOK_TILE,), jnp.inf, jnp.float32)
    ridx = jnp.zeros((TOK_TILE,), jnp.int32)
    for k in range(N_CODE_CHUNKS):
        et = emb_ref[pl.ds(k * CODE_CHUNK, CODE_CHUNK), :]
        mm2 = lax.dot_general(
            xt, et, (((1,), (1,)), ((), ())),
            preferred_element_type=jnp.float32,
        )
        esk = es_ref[0, pl.ds(k * CODE_CHUNK, CODE_CHUNK)]
        t = xs[:, None] + esk[None, :]
        d = t - mm2
        lmin = jnp.min(d, axis=1)
        iot = lax.broadcasted_iota(jnp.int32, (TOK_TILE, CODE_CHUNK), 1)
        lidx = jnp.min(
            jnp.where(d == lmin[:, None], iot, jnp.int32(2**30)), axis=1
        ) + k * CODE_CHUNK
        upd = lmin < rmin
        rmin = jnp.where(upd, lmin, rmin)
        ridx = jnp.where(upd, lidx, ridx)
    codes_ref[0, 0, :] = ridx


def _codes_call(x2b, xs3, es2, emb_b):
    return pl.pallas_call(
        _codes_body,
        grid=(N_TOK_TILES,),
        in_specs=[
            pl.BlockSpec((TOK_TILE, DIM), lambda i: (i, 0)),
            pl.BlockSpec((1, 1, TOK_TILE), lambda i: (i, 0, 0)),
            pl.BlockSpec((1, N_EMB), lambda i: (0, 0)),
            pl.BlockSpec((N_EMB, DIM), lambda i: (0, 0)),
        ],
        out_specs=pl.BlockSpec((1, 1, TOK_TILE), lambda i: (i, 0, 0)),
        out_shape=jax.ShapeDtypeStruct((N_TOK_TILES, 1, TOK_TILE), jnp.int32),
    )(x2b, xs3, es2, emb_b)


def _onehot_body(codes_ref, oh_ref, counts_ref):
    j = pl.program_id(0)
    i = pl.program_id(1)
    c = codes_ref[0, 0, :]
    iot = lax.broadcasted_iota(jnp.int32, (TOK_TILE, OH_TILE_M), 1) \
        + j * OH_TILE_M
    oh = (c[:, None] == iot).astype(jnp.float32)
    oh_ref[...] = oh

    @pl.when(i == 0)
    def _():
        counts_ref[...] = jnp.zeros((1, OH_TILE_M), jnp.float32)

    counts_ref[...] += jnp.sum(oh, axis=0)[None, :]


def _onehot_call(codes3):
    return pl.pallas_call(
        _onehot_body,
        grid=(N_OH_J, N_TOK_TILES),
        in_specs=[
            pl.BlockSpec((1, 1, TOK_TILE), lambda j, i: (i, 0, 0)),
        ],
        out_specs=[
            pl.BlockSpec((TOK_TILE, OH_TILE_M), lambda j, i: (i, j)),
            pl.BlockSpec((1, OH_TILE_M), lambda j, i: (0, j)),
        ],
        out_shape=[
            jax.ShapeDtypeStruct((N_TOK, N_EMB), jnp.float32),
            jax.ShapeDtypeStruct((1, N_EMB), jnp.float32),
        ],
    )(codes3)


def _gather_body(emb_hbm, codes_hbm, x_hbm, out_hbm, lp_hbm,
                 idx_v, q_v, x_v, o_v, lv, sem):
    wid = lax.axis_index("s") * 2 + lax.axis_index("c")
    base = wid * _TOK_PER_W
    pltpu.sync_copy(codes_hbm.at[pl.ds(base, _TOK_PER_W)], idx_v)
    pltpu.async_copy(emb_hbm.at[idx_v], q_v, sem).wait()
    pltpu.sync_copy(x_hbm.at[pl.ds(base, _TOK_PER_W)], x_v)

    def body(t, acc):
        for cc in range(DIM // 16):
            sl = pl.ds(cc * 16, 16)
            xq = x_v[t, sl]
            qq = q_v[t, sl]
            dd = qq - xq
            o_v[t, sl] = xq + dd
            acc = acc + dd * dd
        return acc

    acc = lax.fori_loop(0, _TOK_PER_W, body, jnp.zeros((16,), jnp.float32))
    lv[...] = acc
    pltpu.sync_copy(o_v, out_hbm.at[pl.ds(base, _TOK_PER_W)])
    pltpu.sync_copy(lv, lp_hbm.at[wid])


def _gather_call(emb, codes_flat, x_flat):
    mesh = plsc.VectorSubcoreMesh(core_axis_name="c", subcore_axis_name="s")
    fn = pl.kernel(
        _gather_body,
        out_type=[
            jax.ShapeDtypeStruct((N_TOK, DIM), jnp.float32),
            jax.ShapeDtypeStruct((_SC_WORKERS, 16), jnp.float32),
        ],
        mesh=mesh,
        scratch_types=[
            pltpu.VMEM((_TOK_PER_W,), jnp.int32),
            pltpu.VMEM((_TOK_PER_W, DIM), jnp.float32),
            pltpu.VMEM((_TOK_PER_W, DIM), jnp.float32),
            pltpu.VMEM((_TOK_PER_W, DIM), jnp.float32),
            pltpu.VMEM((16,), jnp.float32),
            pltpu.SemaphoreType.DMA,
        ],
    )
    return fn(emb, codes_flat, x_flat)


def _scalar_body(counts_ref, lp_ref, loss_ref, perp_ref):
    s = jnp.sum(lp_ref[...])
    loss = 0.25 * (s * jnp.float32(1.0 / (N_TOK * DIM)))
    p = counts_ref[0, :] * jnp.float32(1.0 / N_TOK)
    ent = jnp.sum(p * jnp.log(p + 1e-10))
    loss_ref[...] = loss[None, None]
    perp_ref[...] = jnp.exp(-ent)[None, None]


def _scalar_call(counts, lp):
    return pl.pallas_call(
        _scalar_body,
        out_shape=[
            jax.ShapeDtypeStruct((1, 1), jnp.float32),
            jax.ShapeDtypeStruct((1, 1), jnp.float32),
        ],
    )(counts, lp)


def kernel(x, embedding):
    B, T, D = x.shape
    x_flat = x.reshape(-1, D)
    x_sq = jnp.sum(x_flat ** 2, axis=1, keepdims=True)
    e_sq = jnp.sum(embedding ** 2, axis=1)
    xs3 = x_sq.reshape(N_TOK_TILES, 1, TOK_TILE)
    es2 = e_sq.reshape(1, N_EMB)
    x2b = (x_flat + x_flat).astype(jnp.bfloat16)
    emb_b = embedding.astype(jnp.bfloat16)

    codes3 = _codes_call(x2b, xs3, es2, emb_b)
    codes_flat = codes3.reshape(N_TOK)

    one_hot_flat, counts = _onehot_call(codes3)
    quant_st_flat, lp = _gather_call(embedding, codes_flat, x_flat)
    loss2, perp2 = _scalar_call(counts, lp)

    quantized_st = quant_st_flat.reshape(B, T, D)
    codes = codes_flat.reshape(B, T)
    one_hot = one_hot_flat.reshape(B, T, N_EMB)
    return quantized_st, codes, one_hot, loss2[0, 0], perp2[0, 0]
